# Initial kernel scaffold; baseline (speedup 1.0000x reference)
import jax
import jax.numpy as jnp
from jax.experimental import pallas as pl


def kernel(x, edge_index, W1, b1, gamma, beta, W2, b2, eps_param):
    raise NotImplementedError("write your pallas kernel here")


# speedup vs baseline: 7.4791x; 7.4791x over previous
import functools

import jax
import jax.numpy as jnp
from jax.experimental import pallas as pl
from jax.experimental.pallas import tpu as pltpu
from jax.experimental.pallas import tpu_sc as plsc

BN_EPS = 1e-5

NC = 2
NS = 16
LANES = 16
WIN = 80


def _relu_tc(x):
    def body(x_ref, o_ref):
        o_ref[...] = jnp.maximum(x_ref[...], 0.0)

    return pl.pallas_call(
        body, out_shape=jax.ShapeDtypeStruct(x.shape, x.dtype))(x)


def _sc_segment_sum(relu_x, src3d, dst3d, n_pad):
    n, d = relu_x.shape
    windows = src3d.shape[1]
    rows_per_subcore = n_pad // NS
    mesh = plsc.VectorSubcoreMesh(core_axis_name="c", subcore_axis_name="s")

    @functools.partial(
        pl.kernel,
        out_type=jax.ShapeDtypeStruct((NC, n_pad, d), jnp.float32),
        mesh=mesh,
        scratch_types=[
            pltpu.VMEM((windows, WIN), jnp.int32),
            pltpu.VMEM((windows, WIN), jnp.int32),
            pltpu.VMEM((WIN, d), jnp.float32),
            pltpu.VMEM_SHARED((n_pad, d), jnp.float32),
        ],
    )
    def k(relu_x_hbm, src_hbm, dst_hbm, out_hbm, srcv, dstv, buf, acc):
        c = jax.lax.axis_index("c")
        s = jax.lax.axis_index("s")
        wid = c * NS + s

        @pl.loop(0, WIN)
        def _(r):
            @pl.loop(0, d, step=LANES)
            def _(col):
                buf.at[r, pl.ds(col, LANES)][...] = jnp.zeros(
                    (LANES,), jnp.float32)

        row0 = s * rows_per_subcore

        @pl.loop(0, rows_per_subcore, step=WIN)
        def _(r):
            pltpu.sync_copy(buf, acc.at[pl.ds(row0 + r, WIN)])

        pltpu.sync_copy(src_hbm.at[wid], srcv)
        pltpu.sync_copy(dst_hbm.at[wid], dstv)
        plsc.subcore_barrier()

        @pl.loop(0, windows)
        def _(w):
            pltpu.sync_copy(relu_x_hbm.at[srcv.at[w]], buf)
            pltpu.sync_copy(buf, acc.at[dstv.at[w]], add=True)

        plsc.subcore_barrier()
        pltpu.sync_copy(acc.at[pl.ds(row0, rows_per_subcore)],
                        out_hbm.at[c].at[pl.ds(row0, rows_per_subcore)])

    return k(relu_x, src3d, dst3d)


def _mlp_tc(x, parts, W1, b1, gamma, beta, W2, b2, eps_param):
    n, d = x.shape
    h1_dim = W1.shape[1]

    def body(x_ref, p_ref, w1_ref, b1_ref, g_ref, be_ref, w2_ref, b2_ref,
             eps_ref, o_ref):
        h = (x_ref[...] * (1.0 + eps_ref[0, 0])
             + p_ref[0, :n, :] + p_ref[1, :n, :])
        h1 = jnp.dot(h, w1_ref[...],
                     preferred_element_type=jnp.float32) + b1_ref[...]
        mean = jnp.mean(h1, axis=0, keepdims=True)
        var = jnp.mean((h1 - mean) ** 2, axis=0, keepdims=True)
        h1 = (h1 - mean) / jnp.sqrt(var + BN_EPS) * g_ref[...] + be_ref[...]
        h1 = jnp.maximum(h1, 0.0)
        o_ref[...] = jnp.dot(h1, w2_ref[...],
                             preferred_element_type=jnp.float32) + b2_ref[...]

    return pl.pallas_call(
        body,
        out_shape=jax.ShapeDtypeStruct((n, W2.shape[1]), jnp.float32),
    )(x, parts, W1, b1.reshape(1, h1_dim), gamma.reshape(1, h1_dim),
      beta.reshape(1, h1_dim), W2, b2.reshape(1, W2.shape[1]),
      eps_param.reshape(1, 1))


def kernel(x, edge_index, W1, b1, gamma, beta, W2, b2, eps_param):
    n = x.shape[0]
    n_pad = -(-n // (NS * 128)) * (NS * 128)
    relu_x = _relu_tc(x)
    src3d = edge_index[0].reshape(NC * NS, -1, WIN)
    dst3d = edge_index[1].reshape(NC * NS, -1, WIN)
    parts = _sc_segment_sum(relu_x, src3d, dst3d, n_pad)
    return _mlp_tc(x, parts, W1, b1, gamma, beta, W2, b2, eps_param)


# speedup vs baseline: 11.5398x; 1.5429x over previous
import functools

import jax
import jax.numpy as jnp
from jax.experimental import pallas as pl
from jax.experimental.pallas import tpu as pltpu
from jax.experimental.pallas import tpu_sc as plsc

BN_EPS = 1e-5

NC = 2
NS = 16
LANES = 16
WIN = 80


def _relu_tc(x):
    def body(x_ref, o_ref):
        o_ref[...] = jnp.maximum(x_ref[...], 0.0)

    return pl.pallas_call(
        body, out_shape=jax.ShapeDtypeStruct(x.shape, x.dtype))(x)


def _sc_segment_sum(relu_x, packed3d, n_pad):
    n, d = relu_x.shape
    windows = packed3d.shape[1]
    rows_per_subcore = n_pad // NS
    mesh = plsc.VectorSubcoreMesh(core_axis_name="c", subcore_axis_name="s")

    @functools.partial(
        pl.kernel,
        out_type=jax.ShapeDtypeStruct((NC, n_pad, d), jnp.float32),
        mesh=mesh,
        scratch_types=[
            pltpu.VMEM((windows, WIN), jnp.int32),
            pltpu.VMEM((2, 2, WIN), jnp.int32),
            pltpu.VMEM((2, WIN, d), jnp.float32),
            pltpu.VMEM_SHARED((n_pad, d), jnp.float32),
            pltpu.SemaphoreType.DMA,
            pltpu.SemaphoreType.DMA,
            pltpu.SemaphoreType.DMA,
            pltpu.SemaphoreType.DMA,
        ],
    )
    def k(relu_x_hbm, pk_hbm, out_hbm, pk, stage, bufs, acc,
          gs0, gs1, ss0, ss1):
        c = jax.lax.axis_index("c")
        s = jax.lax.axis_index("s")
        wid = c * NS + s
        gsem = (gs0, gs1)
        ssem = (ss0, ss1)

        def unpack(w, p):
            @pl.loop(0, WIN, step=LANES)
            def _(j):
                v = pk.at[w, pl.ds(j, LANES)][...]
                stage.at[p, 0, pl.ds(j, LANES)][...] = (
                    jax.lax.shift_right_logical(v, 15))
                stage.at[p, 1, pl.ds(j, LANES)][...] = (
                    jax.lax.bitwise_and(v, 32767))

        def start_gather(p):
            return pltpu.async_copy(
                relu_x_hbm.at[stage.at[p, 0]], bufs.at[p], gsem[p])

        def wait_gather(p):
            pltpu.make_async_copy(
                relu_x_hbm.at[stage.at[p, 0]], bufs.at[p], gsem[p]).wait()

        def start_scatter(p):
            return pltpu.async_copy(
                bufs.at[p], acc.at[stage.at[p, 1]], ssem[p], add=True)

        def wait_scatter(p):
            pltpu.make_async_copy(
                bufs.at[p], acc.at[stage.at[p, 1]], ssem[p]).wait()

        @pl.loop(0, WIN)
        def _(r):
            @pl.loop(0, d, step=LANES)
            def _(col):
                bufs.at[0, r, pl.ds(col, LANES)][...] = jnp.zeros(
                    (LANES,), jnp.float32)

        row0 = s * rows_per_subcore

        @pl.loop(0, rows_per_subcore, step=WIN)
        def _(r):
            pltpu.sync_copy(bufs.at[0], acc.at[pl.ds(row0 + r, WIN)])

        pltpu.sync_copy(pk_hbm.at[wid], pk)
        plsc.subcore_barrier()

        unpack(0, 0)
        start_gather(0)

        @pl.loop(0, windows, step=2)
        def _(w):
            @pl.when(w > 0)
            def _():
                wait_scatter(1)

            @pl.when(w + 1 < windows)
            def _():
                unpack(w + 1, 1)
                start_gather(1)

            wait_gather(0)
            start_scatter(0)
            wait_scatter(0)

            @pl.when(w + 2 < windows)
            def _():
                unpack(w + 2, 0)
                start_gather(0)

            @pl.when(w + 1 < windows)
            def _():
                wait_gather(1)
                start_scatter(1)

        plsc.subcore_barrier()
        pltpu.sync_copy(acc.at[pl.ds(row0, rows_per_subcore)],
                        out_hbm.at[c].at[pl.ds(row0, rows_per_subcore)])

    return k(relu_x, packed3d)


def _mlp_tc(x, parts, W1, b1, gamma, beta, W2, b2, eps_param):
    n, d = x.shape
    h1_dim = W1.shape[1]

    def body(x_ref, p_ref, w1_ref, b1_ref, g_ref, be_ref, w2_ref, b2_ref,
             eps_ref, o_ref):
        h = (x_ref[...] * (1.0 + eps_ref[0, 0])
             + p_ref[0, :n, :] + p_ref[1, :n, :])
        h1 = jnp.dot(h, w1_ref[...],
                     preferred_element_type=jnp.float32) + b1_ref[...]
        mean = jnp.mean(h1, axis=0, keepdims=True)
        var = jnp.mean((h1 - mean) ** 2, axis=0, keepdims=True)
        h1 = (h1 - mean) / jnp.sqrt(var + BN_EPS) * g_ref[...] + be_ref[...]
        h1 = jnp.maximum(h1, 0.0)
        o_ref[...] = jnp.dot(h1, w2_ref[...],
                             preferred_element_type=jnp.float32) + b2_ref[...]

    return pl.pallas_call(
        body,
        out_shape=jax.ShapeDtypeStruct((n, W2.shape[1]), jnp.float32),
    )(x, parts, W1, b1.reshape(1, h1_dim), gamma.reshape(1, h1_dim),
      beta.reshape(1, h1_dim), W2, b2.reshape(1, W2.shape[1]),
      eps_param.reshape(1, 1))


def kernel(x, edge_index, W1, b1, gamma, beta, W2, b2, eps_param):
    n = x.shape[0]
    n_pad = -(-n // (NS * 128)) * (NS * 128)
    relu_x = _relu_tc(x)
    packed = (edge_index[0] << 15) | edge_index[1]
    packed3d = packed.reshape(NC * NS, -1, WIN)
    parts = _sc_segment_sum(relu_x, packed3d, n_pad)
    return _mlp_tc(x, parts, W1, b1, gamma, beta, W2, b2, eps_param)
